# Initial kernel scaffold; baseline (speedup 1.0000x reference)
import jax
import jax.numpy as jnp
from jax.experimental import pallas as pl


def kernel(vocab_ids, table):
    raise NotImplementedError("write your pallas kernel here")


# speedup vs baseline: 3.8443x; 3.8443x over previous
import jax
import jax.numpy as jnp
from jax import lax
from jax.experimental import pallas as pl
from jax.experimental.pallas import tpu as pltpu
from jax.experimental.pallas import tpu_sc as plsc

VOCAB = 100000
EMBED_DIM = 64
BATCH = 4096
HIST = 50
NUM_IDS = BATCH * HIST

NUM_WORKERS = 32
PER_WORKER = NUM_IDS // NUM_WORKERS
CHUNK = 800
NUM_CHUNKS = PER_WORKER // CHUNK
PAD_DIM = 128


def _sc_gather(table_padded, flat_ids):
    mesh = plsc.VectorSubcoreMesh(core_axis_name="c", subcore_axis_name="s")

    @pl.kernel(
        out_type=jax.ShapeDtypeStruct((NUM_IDS, PAD_DIM), jnp.float32),
        mesh=mesh,
        scratch_types=[
            pltpu.VMEM((CHUNK,), jnp.int32),
            pltpu.VMEM((CHUNK, PAD_DIM), jnp.float32),
            pltpu.SemaphoreType.DMA,
        ],
    )
    def k(table_hbm, ids_hbm, out_hbm, idx_v, rows_v, sem):
        wid = lax.axis_index("s") * 2 + lax.axis_index("c")
        base = wid * PER_WORKER

        @pl.loop(0, NUM_CHUNKS)
        def _(c):
            off = base + c * CHUNK
            pltpu.sync_copy(ids_hbm.at[pl.ds(off, CHUNK)], idx_v)
            pltpu.async_copy(table_hbm.at[idx_v], rows_v, sem).wait()
            pltpu.sync_copy(rows_v, out_hbm.at[pl.ds(off, CHUNK)])

    return k(table_padded, flat_ids)


def kernel(vocab_ids, table):
    flat_ids = vocab_ids.astype(jnp.int32).reshape(NUM_IDS)
    table_padded = jnp.pad(table, ((0, 0), (0, PAD_DIM - EMBED_DIM)))
    out = _sc_gather(table_padded, flat_ids)
    return out[:, :EMBED_DIM].reshape(BATCH, HIST, EMBED_DIM)
